# baseline (speedup 1.0000x reference)
import functools

import jax
import jax.numpy as jnp
from jax import lax
from jax.experimental import pallas as pl
from jax.experimental.pallas import tpu as pltpu
from jax.experimental.pallas import tpu_sc as plsc

B = 16384
EMB = 32
NC = 2
NS = 16
NW = NC * NS
BPW = B // NW
CHUNK = 128
NCHUNK = BPW // CHUNK
RING = 16
L = 16

TABLE_OF = (0, 1, 2, 2, 2, 2, 3, 3, 3)
CONCAT_ORDER = (0, 1, 2, 6, 7, 8, 3, 4, 5)


NB = CHUNK // RING


def _sc_gather_body(user_t, item_t, cate_t, hist_t, idx_hbm, out_hbm,
                    grp_a, grp_b, sel_v, idx_s, sem):
  c = lax.axis_index("c")
  s = lax.axis_index("s")
  wid = s * NC + c
  base = wid * BPW
  tables = (user_t, item_t, cate_t, hist_t)
  grps = (grp_a, grp_b)

  for slot in range(9):
    tab = tables[TABLE_OF[slot]]
    pltpu.sync_copy(idx_hbm.at[slot * NW + wid], idx_s)

    def chunk_body(ci, _, tab=tab, slot=slot):
      def launch(b, buf, tab=tab):
        idx16 = idx_s[ci, pl.ds(b * RING, RING)]
        base16 = idx16 & -8
        for rr in range(RING):
          src = tab.at[pl.ds(pl.multiple_of(base16[rr], 8), 8)]
          pltpu.async_copy(src, buf.at[pl.ds(rr * 8, 8)], sem)

      def drain_select(b, buf, tab=tab):
        pltpu.make_async_copy(tab.at[pl.ds(0, RING * 8)], buf, sem).wait()
        sub16 = idx_s[ci, pl.ds(b * RING, RING)] & 7
        for rr in range(RING):
          sv = sub16[rr]
          r = b * RING + rr
          sel_v[r, pl.ds(0, L)] = buf[rr * 8 + sv, pl.ds(0, L)]
          sel_v[r, pl.ds(L, L)] = buf[rr * 8 + sv, pl.ds(L, L)]

      launch(0, grp_a)

      def pair_body(kb, _2):
        b0 = 2 * kb
        launch(b0 + 1, grp_b)
        drain_select(b0, grp_a)

        @pl.when(kb + 1 < NB // 2)
        def _fire_next():
          launch(b0 + 2, grp_a)

        drain_select(b0 + 1, grp_b)
        return 0

      lax.fori_loop(0, NB // 2, pair_body, 0)
      pltpu.sync_copy(sel_v,
                      out_hbm.at[slot, pl.ds(base + ci * CHUNK, CHUNK)])
      return 0

    lax.fori_loop(0, NCHUNK, chunk_body, 0)


@functools.partial(jax.jit, static_argnames=())
def _sc_gather(user_emb, item_emb, cate_emb, hist_emb, idx9):
  mesh = plsc.VectorSubcoreMesh(core_axis_name="c", subcore_axis_name="s")
  k = pl.kernel(
      _sc_gather_body,
      out_type=jax.ShapeDtypeStruct((9, B, EMB), jnp.float32),
      mesh=mesh,
      scratch_types=[
          pltpu.VMEM((RING * 8, EMB), jnp.float32),
          pltpu.VMEM((RING * 8, EMB), jnp.float32),
          pltpu.VMEM((CHUNK, EMB), jnp.float32),
          pltpu.VMEM((NCHUNK, CHUNK), jnp.int32),
          pltpu.SemaphoreType.DMA,
      ],
  )
  return k(user_emb, item_emb, cate_emb, hist_emb, idx9)


BC = 1024


def _mlp_body(g_ref, n4_ref, w1, b1, w2, b2, w3, b3, w4, b4, out_ref):
  parts = [g_ref[k] for k in CONCAT_ORDER]
  parts.append(n4_ref[...])
  x = jnp.concatenate(parts, axis=1)
  h = jnp.maximum(
      jnp.dot(x, w1[...], preferred_element_type=jnp.float32) + b1[...], 0.0)
  h = jnp.maximum(
      jnp.dot(h, w2[...], preferred_element_type=jnp.float32) + b2[...], 0.0)
  h = jnp.maximum(
      jnp.dot(h, w3[...], preferred_element_type=jnp.float32) + b3[...], 0.0)
  z = jnp.dot(h, w4[...], preferred_element_type=jnp.float32) + b4[...]
  out_ref[...] = 1.0 / (1.0 + jnp.exp(-z))


def _mlp(gath, n4, W1, b1, W2, b2, W3, b3, W4, b4):
  full = lambda shape: pl.BlockSpec(shape, lambda i: (0,) * len(shape))
  return pl.pallas_call(
      _mlp_body,
      grid=(B // BC,),
      in_specs=[
          pl.BlockSpec((9, BC, EMB), lambda i: (0, i, 0)),
          pl.BlockSpec((BC, 4), lambda i: (i, 0)),
          full(W1.shape), full((1, 512)),
          full(W2.shape), full((1, 256)),
          full(W3.shape), full((1, 128)),
          full(W4.shape), full((1, 1)),
      ],
      out_specs=pl.BlockSpec((BC, 1), lambda i: (i, 0)),
      out_shape=jax.ShapeDtypeStruct((B, 1), jnp.float32),
  )(gath, n4, W1, b1, W2, b2, W3, b3, W4, b4)


def kernel(u, i, c, i1, i2, i3, c1, c2, c3, nv, nf, nc, nb,
           user_emb, item_emb, cate_emb, hist_emb,
           W1, b1, W2, b2, W3, b3, W4, b4):
  idx9 = jnp.stack([u, i, c, c1, c2, c3, i1, i2, i3]).astype(jnp.int32)
  idx9 = idx9.reshape(9 * NW, NCHUNK, CHUNK)
  gath = _sc_gather(user_emb, item_emb, cate_emb, hist_emb, idx9)
  n4 = jnp.stack([nv, nf, nc, nb], axis=1)
  out = _mlp(gath, n4,
             W1, b1.reshape(1, -1), W2, b2.reshape(1, -1),
             W3, b3.reshape(1, -1), W4, b4.reshape(1, -1))
  return out[:, 0]


# speedup vs baseline: 1.3935x; 1.3935x over previous
import functools

import jax
import jax.numpy as jnp
from jax import lax
from jax.experimental import pallas as pl
from jax.experimental.pallas import tpu as pltpu
from jax.experimental.pallas import tpu_sc as plsc

B = 16384
EMB = 32
NC = 2
NS = 16
NW = NC * NS
BPW = B // NW
CHUNK = 128
NCHUNK = BPW // CHUNK
RING = 16
L = 16

TABLE_OF = (0, 1, 2, 2, 2, 2, 3, 3, 3)
CONCAT_ORDER = (0, 1, 2, 6, 7, 8, 3, 4, 5)


NB = CHUNK // RING


def _sc_gather_body(user_t, item_t, cate_t, hist_t, idx_hbm, out_hbm,
                    grp_a, grp_b, sel_v, idx_s, sem):
  c = lax.axis_index("c")
  s = lax.axis_index("s")
  wid = s * NC + c
  base = wid * BPW
  tables = (user_t, item_t, cate_t, hist_t)
  grps = (grp_a, grp_b)

  for slot in range(9):
    tab = tables[TABLE_OF[slot]]
    pltpu.sync_copy(idx_hbm.at[slot * NW + wid], idx_s)

    def chunk_body(ci, _, tab=tab, slot=slot):
      def launch(b, buf, tab=tab):
        idx16 = idx_s[ci, pl.ds(b * RING, RING)]
        gi16 = idx16 >> 3
        for rr in range(RING):
          pltpu.async_copy(tab.at[gi16[rr]], buf.at[rr], sem)

      def drain_select(b, buf, tab=tab):
        pltpu.make_async_copy(tab.at[pl.ds(0, RING)], buf, sem).wait()
        sub16 = idx_s[ci, pl.ds(b * RING, RING)] & 7
        for rr in range(RING):
          sv = sub16[rr]
          r = b * RING + rr
          sel_v[r, pl.ds(0, L)] = buf[rr, sv, pl.ds(0, L)]
          sel_v[r, pl.ds(L, L)] = buf[rr, sv, pl.ds(L, L)]

      launch(0, grp_a)

      def pair_body(kb, _2):
        b0 = 2 * kb
        launch(b0 + 1, grp_b)
        drain_select(b0, grp_a)

        @pl.when(kb + 1 < NB // 2)
        def _fire_next():
          launch(b0 + 2, grp_a)

        drain_select(b0 + 1, grp_b)
        return 0

      lax.fori_loop(0, NB // 2, pair_body, 0)
      pltpu.sync_copy(sel_v,
                      out_hbm.at[slot, pl.ds(base + ci * CHUNK, CHUNK)])
      return 0

    lax.fori_loop(0, NCHUNK, chunk_body, 0)


@functools.partial(jax.jit, static_argnames=())
def _sc_gather(user_emb, item_emb, cate_emb, hist_emb, idx9):
  mesh = plsc.VectorSubcoreMesh(core_axis_name="c", subcore_axis_name="s")
  k = pl.kernel(
      _sc_gather_body,
      out_type=jax.ShapeDtypeStruct((9, B, EMB), jnp.float32),
      mesh=mesh,
      scratch_types=[
          pltpu.VMEM((RING, 8, EMB), jnp.float32),
          pltpu.VMEM((RING, 8, EMB), jnp.float32),
          pltpu.VMEM((CHUNK, EMB), jnp.float32),
          pltpu.VMEM((NCHUNK, CHUNK), jnp.int32),
          pltpu.SemaphoreType.DMA,
      ],
      compiler_params=pltpu.CompilerParams(needs_layout_passes=False),
  )
  return k(user_emb.reshape(-1, 8, EMB), item_emb.reshape(-1, 8, EMB),
           cate_emb.reshape(-1, 8, EMB), hist_emb.reshape(-1, 8, EMB),
           idx9)


BC = 1024


def _mlp_body(g_ref, n4_ref, w1, b1, w2, b2, w3, b3, w4, b4, out_ref):
  parts = [g_ref[k] for k in CONCAT_ORDER]
  parts.append(n4_ref[...])
  x = jnp.concatenate(parts, axis=1)
  h = jnp.maximum(
      jnp.dot(x, w1[...], preferred_element_type=jnp.float32) + b1[...], 0.0)
  h = jnp.maximum(
      jnp.dot(h, w2[...], preferred_element_type=jnp.float32) + b2[...], 0.0)
  h = jnp.maximum(
      jnp.dot(h, w3[...], preferred_element_type=jnp.float32) + b3[...], 0.0)
  z = jnp.dot(h, w4[...], preferred_element_type=jnp.float32) + b4[...]
  out_ref[...] = 1.0 / (1.0 + jnp.exp(-z))


def _mlp(gath, n4, W1, b1, W2, b2, W3, b3, W4, b4):
  full = lambda shape: pl.BlockSpec(shape, lambda i: (0,) * len(shape))
  return pl.pallas_call(
      _mlp_body,
      grid=(B // BC,),
      in_specs=[
          pl.BlockSpec((9, BC, EMB), lambda i: (0, i, 0)),
          pl.BlockSpec((BC, 4), lambda i: (i, 0)),
          full(W1.shape), full((1, 512)),
          full(W2.shape), full((1, 256)),
          full(W3.shape), full((1, 128)),
          full(W4.shape), full((1, 1)),
      ],
      out_specs=pl.BlockSpec((BC, 1), lambda i: (i, 0)),
      out_shape=jax.ShapeDtypeStruct((B, 1), jnp.float32),
  )(gath, n4, W1, b1, W2, b2, W3, b3, W4, b4)


def kernel(u, i, c, i1, i2, i3, c1, c2, c3, nv, nf, nc, nb,
           user_emb, item_emb, cate_emb, hist_emb,
           W1, b1, W2, b2, W3, b3, W4, b4):
  idx9 = jnp.stack([u, i, c, c1, c2, c3, i1, i2, i3]).astype(jnp.int32)
  idx9 = idx9.reshape(9 * NW, NCHUNK, CHUNK)
  gath = _sc_gather(user_emb, item_emb, cate_emb, hist_emb, idx9)
  n4 = jnp.stack([nv, nf, nc, nb], axis=1)
  out = _mlp(gath, n4,
             W1, b1.reshape(1, -1), W2, b2.reshape(1, -1),
             W3, b3.reshape(1, -1), W4, b4.reshape(1, -1))
  return out[:, 0]
